# baseline (speedup 1.0000x reference)
import functools

import jax
import jax.numpy as jnp
from jax import lax
from jax.experimental import pallas as pl
from jax.experimental.pallas import tpu as pltpu
from jax.experimental.pallas import tpu_sc as plsc

D_MODEL = 64
MAX_LEN = 512
NUM_CORES = 2
NUM_SUBCORES = 16
NUM_WORKERS = NUM_CORES * NUM_SUBCORES

CHUNK = 128
GROUPS = CHUNK // 16


def _pos_encoding():
    pos = jnp.arange(MAX_LEN, dtype=jnp.float32)[:, None]
    _2i = jnp.arange(0, D_MODEL, 2, dtype=jnp.float32)
    ang = pos / jnp.power(10000.0, _2i / D_MODEL)
    pe = jnp.zeros((MAX_LEN, D_MODEL), dtype=jnp.float32)
    pe = pe.at[:, 0::2].set(jnp.sin(ang))
    pe = pe.at[:, 1::2].set(jnp.cos(ang))
    return pe


@jax.jit
def _embed(x, weight):
    batch, seq = x.shape
    x2 = x.reshape(-1, CHUNK)
    w128 = weight.reshape(-1, 128)
    pe_t = _pos_encoding().T
    n_chunks = x2.shape[0]
    per_w = n_chunks // NUM_WORKERS
    sub = seq // CHUNK
    mesh = plsc.VectorSubcoreMesh(core_axis_name="c", subcore_axis_name="s")

    @functools.partial(
        pl.kernel,
        out_type=jax.ShapeDtypeStruct((batch, D_MODEL, seq), jnp.float32),
        mesh=mesh,
        compiler_params=pltpu.CompilerParams(needs_layout_passes=False),
        scratch_types=[
            pltpu.VMEM((D_MODEL, MAX_LEN), jnp.float32),
            pltpu.VMEM((2, CHUNK), jnp.int32),
            pltpu.VMEM((2, CHUNK), jnp.int32),
            pltpu.VMEM((2, CHUNK, 128), jnp.float32),
            pltpu.VMEM((2, D_MODEL, CHUNK), jnp.float32),
            pltpu.SemaphoreType.DMA,
            pltpu.SemaphoreType.DMA,
            pltpu.SemaphoreType.DMA,
            pltpu.SemaphoreType.DMA,
            pltpu.SemaphoreType.DMA,
            pltpu.SemaphoreType.DMA,
        ],
    )
    def kern(x_hbm, w_hbm, pe_hbm, out_hbm, pe_v, idx_v, pair_v, rows_v,
             blk_v, sg0, sg1, si0, si1, so0, so1):
        sem_g = (sg0, sg1)
        sem_i = (si0, si1)
        sem_o = (so0, so1)
        wid = lax.axis_index("s") * NUM_CORES + lax.axis_index("c")
        pltpu.sync_copy(pe_hbm, pe_v)
        c0 = wid * per_w

        def load_idx(ci, buf, sem):
            return pltpu.async_copy(x_hbm.at[c0 + ci], idx_v.at[buf], sem)

        def fire_gather(buf, sem):
            for g in range(GROUPS):
                sl = pl.ds(g * 16, 16)
                pair_v[buf, sl] = lax.shift_right_logical(idx_v[buf, sl], 1)
            pltpu.async_copy(
                w_hbm.at[pair_v.at[buf]], rows_v.at[buf], sem,
            )

        def drain_gather(buf, sem):
            pltpu.make_async_copy(
                w_hbm.at[pair_v.at[buf]], rows_v.at[buf], sem,
            ).wait()

        def out_slice(ci):
            gc = c0 + ci
            return out_hbm.at[gc // sub].at[:, pl.ds((gc % sub) * CHUNK, CHUNK)]

        load_idx(0, 0, sem_i[0]).wait()
        fire_gather(0, sem_g[0])
        load_idx(1, 1, sem_i[1])

        @pl.loop(0, per_w, step=2)
        def _(c):
            for b in range(2):
                cc = c + b
                o = 1 - b
                drain_gather(b, sem_g[b])

                @pl.when(cc > 0)
                def _():
                    pltpu.make_async_copy(
                        blk_v.at[o], out_slice(cc - 1), sem_o[o],
                    ).wait()

                @pl.when(cc + 1 < per_w)
                def _():
                    pltpu.make_async_copy(
                        x_hbm.at[c0 + cc + 1], idx_v.at[o], sem_i[o],
                    ).wait()
                    fire_gather(o, sem_g[o])

                s0 = (cc % sub) * CHUNK
                iot = lax.iota(jnp.int32, 16)
                for g in range(GROUPS):
                    sl = pl.ds(g * 16, 16)
                    h64 = (idx_v[b, sl] & 1) * 64
                    row = iot + g * 16
                    svec = iot + (s0 + g * 16)

                    @pl.loop(0, 16)
                    def _(k):
                        dmod = (iot + k) & 15
                        for c16 in range(D_MODEL // 16):
                            d_vec = dmod + c16 * 16
                            vals = plsc.load_gather(
                                rows_v.at[b], [row, h64 + d_vec]
                            )
                            pev = plsc.load_gather(pe_v, [d_vec, svec])
                            plsc.store_scatter(
                                blk_v.at[b], [d_vec, row], vals + pev
                            )

                @pl.when(cc + 2 < per_w)
                def _():
                    load_idx(cc + 2, b, sem_i[b])

                pltpu.async_copy(blk_v.at[b], out_slice(cc), sem_o[b])

        pltpu.make_async_copy(
            blk_v.at[(per_w - 1) % 2],
            out_slice(per_w - 1),
            sem_o[(per_w - 1) % 2],
        ).wait()

    return kern(x2, w128, pe_t)


def kernel(x, weight):
    return _embed(x, weight).transpose(0, 2, 1)
